# baseline (speedup 1.0000x reference)
import functools
import jax
import jax.numpy as jnp
from jax.experimental import pallas as pl
from jax.experimental.pallas import tpu as pltpu

N = 10000
D = 128
HID = 32
FMAX = jnp.float32(3.0e38)


def _stage_b_body(x_ref, ssum_ref, ssq_ref, mx_ref, mn_ref, deg_ref, delta_ref,
                  wp0_ref, wpa_ref, wpb_ref, wpc_ref, bpost_ref, w1a_ref, w1b_ref,
                  u_ref, v_ref):
    x = x_ref[...]
    deg = deg_ref[...]
    degc = jnp.maximum(deg, 1.0)
    mean = ssum_ref[...] / degc
    var = ssq_ref[...] / degc - mean * mean
    std = jnp.sqrt(jnp.maximum(var, 0.0) + 1e-5)
    pos = deg > 0.0
    mx = jnp.where(pos, mx_ref[...], 0.0)
    mn = jnp.where(pos, mn_ref[...], 0.0)
    A = jnp.concatenate([mean, mx, mn, std], axis=1)
    logd = jnp.log(deg + 1.0)
    delta = delta_ref[...]
    amp = logd / delta
    att = jnp.where(logd > 0.0, delta / jnp.maximum(logd, 1e-6), 1.0)
    f32 = jnp.float32
    h_pre = (jnp.dot(x, wp0_ref[...], preferred_element_type=f32)
             + jnp.dot(A, wpa_ref[...], preferred_element_type=f32)
             + jnp.dot(A * amp, wpb_ref[...], preferred_element_type=f32)
             + jnp.dot(A * att, wpc_ref[...], preferred_element_type=f32))
    h = jnp.maximum(h_pre + bpost_ref[...], 0.0) + x
    u_ref[...] = jnp.dot(h, w1a_ref[...], preferred_element_type=f32)
    v_ref[...] = jnp.dot(h, w1b_ref[...], preferred_element_type=f32)


def _stage_b(x, ssum, ssq, mx, mn, deg, delta, W_post, b_post, W1):
    BN = 400
    grid = (N // BN,)
    full = lambda shape: pl.BlockSpec(shape, lambda i: (0, 0))
    blk = lambda cols: pl.BlockSpec((BN, cols), lambda i: (i, 0))
    Wp0 = W_post[:D]
    WpA = W_post[D:5 * D]
    WpB = W_post[5 * D:9 * D]
    WpC = W_post[9 * D:]
    return pl.pallas_call(
        _stage_b_body,
        grid=grid,
        in_specs=[blk(D), blk(D), blk(D), blk(D), blk(D), blk(1), full((1, 1)),
                  full((D, D)), full((4 * D, D)), full((4 * D, D)), full((4 * D, D)),
                  full((1, D)), full((D, HID)), full((D, HID))],
        out_specs=[blk(HID), blk(HID)],
        out_shape=[jax.ShapeDtypeStruct((N, HID), jnp.float32),
                   jax.ShapeDtypeStruct((N, HID), jnp.float32)],
    )(x, ssum, ssq, mx, mn, deg, delta, Wp0, WpA, WpB, WpC,
      b_post.reshape(1, D), W1[:D], W1[D:])


def _delta_body(deg_ref, out_ref):
    out_ref[...] = (jnp.sum(jnp.log(deg_ref[...] + 1.0)) / N).reshape(1, 1)


def _delta(deg):
    return pl.pallas_call(
        _delta_body,
        in_specs=[pl.BlockSpec((N, 1), lambda: (0, 0))],
        out_specs=pl.BlockSpec((1, 1), lambda: (0, 0)),
        out_shape=jax.ShapeDtypeStruct((1, 1), jnp.float32),
    )(deg)


def _stage_d_body(z_ref, mom_ref, gamma_ref, beta_ref, w2_ref, b2_ref, out_ref, *, P):
    mom = mom_ref[...]
    mu = mom[0:1] / P
    var = mom[1:2] / P - mu * mu
    inv = jax.lax.rsqrt(var + 1e-5)
    zn = (z_ref[...] - mu) * (inv * gamma_ref[...]) + beta_ref[...]
    zn = jnp.maximum(zn, 0.0)
    out_ref[...] = (jnp.dot(zn, w2_ref[...], preferred_element_type=jnp.float32)
                    + b2_ref[...])


def _stage_d(z, moments, gamma, beta, W2, b2):
    P = z.shape[0]
    BP = 2000
    grid = (P // BP,)
    full = lambda shape: pl.BlockSpec(shape, lambda i: (0, 0))
    return pl.pallas_call(
        functools.partial(_stage_d_body, P=P),
        grid=grid,
        in_specs=[pl.BlockSpec((BP, HID), lambda i: (i, 0)), full((2, HID)),
                  full((1, HID)), full((1, HID)), full((HID, 1)), full((1, 1))],
        out_specs=pl.BlockSpec((BP, 1), lambda i: (i, 0)),
        out_shape=jax.ShapeDtypeStruct((P, 1), jnp.float32),
    )(z, moments, gamma.reshape(1, HID), beta.reshape(1, HID), W2, b2.reshape(1, 1))


def _stage_a_jax(x, src, dst):
    m = x[src]
    deg = jax.ops.segment_sum(jnp.ones(src.shape, jnp.float32), dst, num_segments=N)
    ssum = jax.ops.segment_sum(m, dst, num_segments=N)
    ssq = jax.ops.segment_sum(m * m, dst, num_segments=N)
    mx = jax.ops.segment_max(m, dst, num_segments=N)
    mn = -jax.ops.segment_max(-m, dst, num_segments=N)
    return deg.reshape(N, 1), ssum, ssq, mx, mn


def _stage_c_jax(u, v, pi0, pi1):
    z = u[pi0] + v[pi1]
    moments = jnp.stack([jnp.sum(z, axis=0), jnp.sum(z * z, axis=0)])
    return z, moments


def kernel(x, edge_index, pairwise_indices, W_post, b_post, W1, b1, gamma, beta, W2, b2):
    src = edge_index[0]
    dst = edge_index[1]
    deg, ssum, ssq, mx, mn = _stage_a_jax(x, src, dst)
    delta = _delta(deg)
    u, v = _stage_b(x, ssum, ssq, mx, mn, deg, delta, W_post, b_post, W1)
    z, moments = _stage_c_jax(u, v, pairwise_indices[0], pairwise_indices[1])
    return _stage_d(z, moments, gamma, beta, W2, b2)


# speedup vs baseline: 1.1726x; 1.1726x over previous
import functools
import jax
import jax.numpy as jnp
from jax import lax
from jax.experimental import pallas as pl
from jax.experimental.pallas import tpu as pltpu
from jax.experimental.pallas import tpu_sc as plsc

N = 10000
D = 128
HID = 32
P = 320000
NW = 32
PW = P // NW
GC = 80
NCH = PW // GC
FMAX = 3.0e38


def _sc_mesh():
    return plsc.VectorSubcoreMesh(core_axis_name="c", subcore_axis_name="s",
                                  num_cores=2, num_subcores=16)


def _stage_b_body(x_ref, ssum_ref, ssq_ref, mx_ref, mn_ref, deg_ref, delta_ref,
                  wp0_ref, wpa_ref, wpb_ref, wpc_ref, bpost_ref, w1p_ref,
                  t_ref):
    x = x_ref[...]
    deg = deg_ref[...]
    degc = jnp.maximum(deg, 1.0)
    mean = ssum_ref[...] / degc
    var = ssq_ref[...] / degc - mean * mean
    std = jnp.sqrt(jnp.maximum(var, 0.0) + 1e-5)
    pos = deg > 0.0
    mx = jnp.where(pos, mx_ref[...], 0.0)
    mn = jnp.where(pos, mn_ref[...], 0.0)
    A = jnp.concatenate([mean, mx, mn, std], axis=1)
    logd = jnp.log(deg + 1.0)
    delta = delta_ref[...]
    amp = logd / delta
    att = jnp.where(logd > 0.0, delta / jnp.maximum(logd, 1e-6), 1.0)
    f32 = jnp.float32
    h_pre = (jnp.dot(x, wp0_ref[...], preferred_element_type=f32)
             + jnp.dot(A, wpa_ref[...], preferred_element_type=f32)
             + jnp.dot(A * amp, wpb_ref[...], preferred_element_type=f32)
             + jnp.dot(A * att, wpc_ref[...], preferred_element_type=f32))
    h = jnp.maximum(h_pre + bpost_ref[...], 0.0) + x
    t_ref[...] = jnp.dot(h, w1p_ref[...], preferred_element_type=f32)


def _stage_b(x, ssum, ssq, mx, mn, deg, delta, W_post, b_post, W1):
    BN = 400
    grid = (N // BN,)
    full = lambda shape: pl.BlockSpec(shape, lambda i: (0, 0))
    blk = lambda cols: pl.BlockSpec((BN, cols), lambda i: (i, 0))
    Wp0 = W_post[:D]
    WpA = W_post[D:5 * D]
    WpB = W_post[5 * D:9 * D]
    WpC = W_post[9 * D:]
    W1p = jnp.concatenate([W1[:D], W1[D:], jnp.zeros((D, D - 2 * HID), jnp.float32)],
                          axis=1)
    return pl.pallas_call(
        _stage_b_body,
        grid=grid,
        in_specs=[blk(D), blk(D), blk(D), blk(D), blk(D), blk(1), full((1, 1)),
                  full((D, D)), full((4 * D, D)), full((4 * D, D)), full((4 * D, D)),
                  full((1, D)), full((D, D))],
        out_specs=blk(D),
        out_shape=jax.ShapeDtypeStruct((N, D), jnp.float32),
    )(x, ssum, ssq, mx, mn, deg, delta, Wp0, WpA, WpB, WpC,
      b_post.reshape(1, D), W1p)


def _delta_body(deg_ref, out_ref):
    out_ref[...] = (jnp.sum(jnp.log(deg_ref[...] + 1.0)) / N).reshape(1, 1)


def _delta(deg):
    return pl.pallas_call(
        _delta_body,
        in_specs=[pl.BlockSpec((N, 1), lambda: (0, 0))],
        out_specs=pl.BlockSpec((1, 1), lambda: (0, 0)),
        out_shape=jax.ShapeDtypeStruct((1, 1), jnp.float32),
    )(deg)


def _stage_d_body(z_ref, mom_ref, gamma_ref, beta_ref, w2_ref, b2_ref, out_ref, *, P):
    part = mom_ref[...]
    mu = jnp.sum(part[:, :HID], axis=0, keepdims=True) / P
    var = jnp.sum(part[:, HID:], axis=0, keepdims=True) / P - mu * mu
    inv = jax.lax.rsqrt(var + 1e-5)
    zn = (z_ref[...] - mu) * (inv * gamma_ref[...]) + beta_ref[...]
    zn = jnp.maximum(zn, 0.0)
    out_ref[...] = (jnp.dot(zn, w2_ref[...], preferred_element_type=jnp.float32)
                    + b2_ref[...])


def _stage_d(z, moments, gamma, beta, W2, b2):
    BP = 2000
    grid = (P // BP,)
    full = lambda shape: pl.BlockSpec(shape, lambda i: (0, 0))
    return pl.pallas_call(
        functools.partial(_stage_d_body, P=P),
        grid=grid,
        in_specs=[pl.BlockSpec((BP, HID), lambda i: (i, 0)), full((NW, 2 * HID)),
                  full((1, HID)), full((1, HID)), full((HID, 1)), full((1, 1))],
        out_specs=pl.BlockSpec((BP, 1), lambda i: (i, 0)),
        out_shape=jax.ShapeDtypeStruct((P, 1), jnp.float32),
    )(z, moments, gamma.reshape(1, HID), beta.reshape(1, HID), W2, b2.reshape(1, 1))


def _stage_a_jax(x, src, dst):
    m = x[src]
    deg = jax.ops.segment_sum(jnp.ones(src.shape, jnp.float32), dst, num_segments=N)
    ssum = jax.ops.segment_sum(m, dst, num_segments=N)
    ssq = jax.ops.segment_sum(m * m, dst, num_segments=N)
    mx = jax.ops.segment_max(m, dst, num_segments=N)
    mn = -jax.ops.segment_max(-m, dst, num_segments=N)
    return deg.reshape(N, 1), ssum, ssq, mx, mn


def _stage_c_body(t_hbm, pi0_hbm, pi1_hbm, z_hbm, part_hbm,
                  idx0, idx1, urows, vrows, zbuf, pbuf, sem):
    c = lax.axis_index("c")
    s = lax.axis_index("s")
    wid = s * 2 + c
    base = wid * PW
    zero = jnp.zeros((16,), jnp.float32)

    def chunk(j, carry):
        s0, s1, q0, q1 = carry
        off = base + j * GC
        pltpu.sync_copy(pi0_hbm.at[pl.ds(off, GC)], idx0)
        pltpu.sync_copy(pi1_hbm.at[pl.ds(off, GC)], idx1)
        cp1 = pltpu.async_copy(t_hbm.at[idx0], urows, sem)
        cp2 = pltpu.async_copy(t_hbm.at[idx1], vrows, sem)
        cp1.wait()
        cp2.wait()

        def row(i, rc):
            rs0, rs1, rq0, rq1 = rc
            a0 = urows[i, pl.ds(0, 16)] + vrows[i, pl.ds(32, 16)]
            a1 = urows[i, pl.ds(16, 16)] + vrows[i, pl.ds(48, 16)]
            zbuf[i, pl.ds(0, 16)] = a0
            zbuf[i, pl.ds(16, 16)] = a1
            return (rs0 + a0, rs1 + a1, rq0 + a0 * a0, rq1 + a1 * a1)

        carry = lax.fori_loop(0, GC, row, (s0, s1, q0, q1))
        pltpu.sync_copy(zbuf, z_hbm.at[pl.ds(off, GC)])
        return carry

    s0, s1, q0, q1 = lax.fori_loop(0, NCH, chunk, (zero, zero, zero, zero))
    pbuf[pl.ds(0, 16)] = s0
    pbuf[pl.ds(16, 16)] = s1
    pbuf[pl.ds(32, 16)] = q0
    pbuf[pl.ds(48, 16)] = q1
    pltpu.sync_copy(pbuf, part_hbm.at[wid])


def _stage_c(t, pi0, pi1):
    f = pl.kernel(
        _stage_c_body,
        out_type=[jax.ShapeDtypeStruct((P, HID), jnp.float32),
                  jax.ShapeDtypeStruct((NW, 2 * HID), jnp.float32)],
        mesh=_sc_mesh(),
        scratch_types=[pltpu.VMEM((GC,), jnp.int32), pltpu.VMEM((GC,), jnp.int32),
                       pltpu.VMEM((GC, D), jnp.float32),
                       pltpu.VMEM((GC, D), jnp.float32),
                       pltpu.VMEM((GC, HID), jnp.float32),
                       pltpu.VMEM((2 * HID,), jnp.float32),
                       pltpu.SemaphoreType.DMA],
    )
    return f(t, pi0, pi1)


def kernel(x, edge_index, pairwise_indices, W_post, b_post, W1, b1, gamma, beta, W2, b2):
    src = edge_index[0]
    dst = edge_index[1]
    deg, ssum, ssq, mx, mn = _stage_a_jax(x, src, dst)
    delta = _delta(deg)
    t = _stage_b(x, ssum, ssq, mx, mn, deg, delta, W_post, b_post, W1)
    z, moments = _stage_c(t, pairwise_indices[0], pairwise_indices[1])
    return _stage_d(z, moments, gamma, beta, W2, b2)


# speedup vs baseline: 2.0129x; 1.7165x over previous
import functools
import jax
import jax.numpy as jnp
from jax import lax
from jax.experimental import pallas as pl
from jax.experimental.pallas import tpu as pltpu
from jax.experimental.pallas import tpu_sc as plsc

N = 10000
D = 128
HID = 32
P = 320000
NW = 32
PW = P // NW
GC = 80
NCH = PW // GC
FMAX = 3.0e38


def _sc_mesh():
    return plsc.VectorSubcoreMesh(core_axis_name="c", subcore_axis_name="s",
                                  num_cores=2, num_subcores=16)


def _stage_b_body(x_ref, ssum_ref, ssq_ref, mx_ref, mn_ref, deg_ref, delta_ref,
                  wp0_ref, wpa_ref, wpb_ref, wpc_ref, bpost_ref, w1p_ref,
                  t_ref):
    x = x_ref[...]
    deg = deg_ref[...]
    degc = jnp.maximum(deg, 1.0)
    mean = ssum_ref[...] / degc
    var = ssq_ref[...] / degc - mean * mean
    std = jnp.sqrt(jnp.maximum(var, 0.0) + 1e-5)
    pos = deg > 0.0
    mx = jnp.where(pos, mx_ref[...], 0.0)
    mn = jnp.where(pos, mn_ref[...], 0.0)
    A = jnp.concatenate([mean, mx, mn, std], axis=1)
    logd = jnp.log(deg + 1.0)
    delta = delta_ref[...]
    amp = logd / delta
    att = jnp.where(logd > 0.0, delta / jnp.maximum(logd, 1e-6), 1.0)
    f32 = jnp.float32
    h_pre = (jnp.dot(x, wp0_ref[...], preferred_element_type=f32)
             + jnp.dot(A, wpa_ref[...], preferred_element_type=f32)
             + jnp.dot(A * amp, wpb_ref[...], preferred_element_type=f32)
             + jnp.dot(A * att, wpc_ref[...], preferred_element_type=f32))
    h = jnp.maximum(h_pre + bpost_ref[...], 0.0) + x
    t_ref[...] = jnp.dot(h, w1p_ref[...], preferred_element_type=f32)


def _stage_b(x, ssum, ssq, mx, mn, deg, delta, W_post, b_post, W1):
    BN = 400
    grid = (N // BN,)
    full = lambda shape: pl.BlockSpec(shape, lambda i: (0, 0))
    blk = lambda cols: pl.BlockSpec((BN, cols), lambda i: (i, 0))
    Wp0 = W_post[:D]
    WpA = W_post[D:5 * D]
    WpB = W_post[5 * D:9 * D]
    WpC = W_post[9 * D:]
    W1p = jnp.concatenate([W1[:D], W1[D:], jnp.zeros((D, D - 2 * HID), jnp.float32)],
                          axis=1)
    return pl.pallas_call(
        _stage_b_body,
        grid=grid,
        in_specs=[blk(D), blk(D), blk(D), blk(D), blk(D), blk(1), full((1, 1)),
                  full((D, D)), full((4 * D, D)), full((4 * D, D)), full((4 * D, D)),
                  full((1, D)), full((D, D))],
        out_specs=blk(D),
        out_shape=jax.ShapeDtypeStruct((N, D), jnp.float32),
    )(x, ssum, ssq, mx, mn, deg, delta, Wp0, WpA, WpB, WpC,
      b_post.reshape(1, D), W1p)


def _delta_body(deg_ref, out_ref):
    out_ref[...] = (jnp.sum(jnp.log(deg_ref[...] + 1.0)) / N).reshape(1, 1)


def _delta(deg):
    return pl.pallas_call(
        _delta_body,
        in_specs=[pl.BlockSpec((N, 1), lambda: (0, 0))],
        out_specs=pl.BlockSpec((1, 1), lambda: (0, 0)),
        out_shape=jax.ShapeDtypeStruct((1, 1), jnp.float32),
    )(deg)


def _stage_d_body(z_ref, mom_ref, gamma_ref, beta_ref, w2_ref, b2_ref, out_ref, *, P):
    part = mom_ref[...]
    mu = jnp.sum(part[:, :HID], axis=0, keepdims=True) / P
    var = jnp.sum(part[:, HID:], axis=0, keepdims=True) / P - mu * mu
    inv = jax.lax.rsqrt(var + 1e-5)
    zn = (z_ref[...] - mu) * (inv * gamma_ref[...]) + beta_ref[...]
    zn = jnp.maximum(zn, 0.0)
    out_ref[...] = (jnp.dot(zn, w2_ref[...], preferred_element_type=jnp.float32)
                    + b2_ref[...])


def _stage_d(z, moments, gamma, beta, W2, b2):
    BP = 2000
    grid = (P // BP,)
    full = lambda shape: pl.BlockSpec(shape, lambda i: (0, 0))
    return pl.pallas_call(
        functools.partial(_stage_d_body, P=P),
        grid=grid,
        in_specs=[pl.BlockSpec((BP, HID), lambda i: (i, 0)), full((NW, 2 * HID)),
                  full((1, HID)), full((1, HID)), full((HID, 1)), full((1, 1))],
        out_specs=pl.BlockSpec((BP, 1), lambda i: (i, 0)),
        out_shape=jax.ShapeDtypeStruct((P, 1), jnp.float32),
    )(z, moments, gamma.reshape(1, HID), beta.reshape(1, HID), W2, b2.reshape(1, 1))


EE = 320000
ECH = 2560
ESUB = 64
ENSUB = ECH // ESUB
ENCHK = EE // ECH
NBA = 320
ETRASH = 144


def _stage_a_body(dst_hbm, src_hbm, x_hbm,
                  ssum_hbm, ssq_hbm, mx_hbm, mn_hbm, deg_hbm,
                  dchunk, schunk, pend_dl, pend_src, rows, acc1, acc2, dega,
                  sem):
    c = lax.axis_index("c")
    s = lax.axis_index("s")
    r = s * 2 + c
    lo = r * NBA
    hi = lo + NBA
    i16 = lax.iota(jnp.int32, 16)
    e1 = (i16 == 0).astype(jnp.float32)
    zero_f = jnp.zeros((16,), jnp.float32)
    zero_i = jnp.zeros((16,), jnp.int32)

    def init_phase(phase):
        v1 = zero_f if phase == 0 else jnp.full((16,), -FMAX, jnp.float32)
        v2 = zero_f if phase == 0 else jnp.full((16,), FMAX, jnp.float32)

        def row(i, carry):
            for cv in range(8):
                acc1[i, pl.ds(cv * 16, 16)] = v1
                acc2[i, pl.ds(cv * 16, 16)] = v2
            if phase == 0:
                dega[pl.ds(i * 16, 16)] = zero_f
            return carry

        lax.fori_loop(0, NBA + 1, row, 0)

    def accum_batch(phase):
        cp = pltpu.async_copy(x_hbm.at[pend_src.at[pl.ds(0, ESUB)]], rows, sem)
        cp.wait()

        def acc_group(g, carry):
            dlv = pend_dl[pl.ds(g * 16, 16)]
            for t in range(16):
                dl = dlv[t]
                e = g * 16 + t
                for cv in range(8):
                    rv = rows[e, pl.ds(cv * 16, 16)]
                    a1 = acc1[dl, pl.ds(cv * 16, 16)]
                    a2 = acc2[dl, pl.ds(cv * 16, 16)]
                    if phase == 0:
                        acc1[dl, pl.ds(cv * 16, 16)] = a1 + rv
                        acc2[dl, pl.ds(cv * 16, 16)] = a2 + rv * rv
                    else:
                        acc1[dl, pl.ds(cv * 16, 16)] = jnp.maximum(a1, rv)
                        acc2[dl, pl.ds(cv * 16, 16)] = jnp.minimum(a2, rv)
                if phase == 0:
                    dv = dega[pl.ds(dl * 16, 16)]
                    dega[pl.ds(dl * 16, 16)] = dv + e1
            return carry

        lax.fori_loop(0, ESUB // 16, acc_group, 0)

    def shift_residual():
        for w in range(ESUB // 16):
            v_dl = pend_dl[pl.ds(ESUB + w * 16, 16)]
            v_sr = pend_src[pl.ds(ESUB + w * 16, 16)]
            pend_dl[pl.ds(w * 16, 16)] = v_dl
            pend_src[pl.ds(w * 16, 16)] = v_sr

    def run_phase(phase):
        init_phase(phase)

        def chunk(j, np_vec):
            pltpu.sync_copy(dst_hbm.at[pl.ds(j * ECH, ECH)], dchunk)
            pltpu.sync_copy(src_hbm.at[pl.ds(j * ECH, ECH)], schunk)

            def sub(si, np_vec):
                boff = si * ESUB
                for k in range(ESUB // 16):
                    dstv = dchunk[pl.ds(boff + k * 16, 16)]
                    srcv = schunk[pl.ds(boff + k * 16, 16)]
                    m = (dstv >= lo) & (dstv < hi)
                    rank = plsc.cumsum(m.astype(jnp.int32)) - 1
                    idx = jnp.where(m, np_vec + rank, ETRASH)
                    plsc.store_scatter(pend_dl, [idx], dstv - lo)
                    plsc.store_scatter(pend_src, [idx], srcv)
                    np_vec = np_vec + plsc.all_reduce_population_count(m)
                nps = lax.reduce_max(np_vec, axes=(0,))
                do = nps >= ESUB

                @pl.when(do)
                def _():
                    accum_batch(phase)
                    shift_residual()

                return jnp.where(do, np_vec - ESUB, np_vec)

            return lax.fori_loop(0, ENSUB, sub, np_vec)

        np_vec = lax.fori_loop(0, ENCHK, chunk, zero_i)
        for w in range(ESUB // 16):
            pos = i16 + w * 16
            keep = pos < np_vec
            cur_dl = pend_dl[pl.ds(w * 16, 16)]
            cur_sr = pend_src[pl.ds(w * 16, 16)]
            pend_dl[pl.ds(w * 16, 16)] = jnp.where(keep, cur_dl, NBA)
            pend_src[pl.ds(w * 16, 16)] = jnp.where(keep, cur_sr, (r * 311) % N)
        accum_batch(phase)

        def write(nrows):
            row0 = r * NBA
            if phase == 0:
                pltpu.sync_copy(acc1.at[pl.ds(0, nrows)], ssum_hbm.at[pl.ds(row0, nrows)])
                pltpu.sync_copy(acc2.at[pl.ds(0, nrows)], ssq_hbm.at[pl.ds(row0, nrows)])
                pltpu.sync_copy(dega.at[pl.ds(0, nrows * 16)],
                                deg_hbm.at[pl.ds(row0 * 16, nrows * 16)])
            else:
                pltpu.sync_copy(acc1.at[pl.ds(0, nrows)], mx_hbm.at[pl.ds(row0, nrows)])
                pltpu.sync_copy(acc2.at[pl.ds(0, nrows)], mn_hbm.at[pl.ds(row0, nrows)])

        @pl.when(r < 31)
        def _():
            write(NBA)

        @pl.when(r == 31)
        def _():
            write(N - 31 * NBA)

    run_phase(0)
    run_phase(1)


def _stage_a(dst, src, x):
    f = pl.kernel(
        _stage_a_body,
        out_type=[jax.ShapeDtypeStruct((N, D), jnp.float32),
                  jax.ShapeDtypeStruct((N, D), jnp.float32),
                  jax.ShapeDtypeStruct((N, D), jnp.float32),
                  jax.ShapeDtypeStruct((N, D), jnp.float32),
                  jax.ShapeDtypeStruct((N * 16,), jnp.float32)],
        mesh=_sc_mesh(),
        scratch_types=[pltpu.VMEM((ECH,), jnp.int32), pltpu.VMEM((ECH,), jnp.int32),
                       pltpu.VMEM((160,), jnp.int32), pltpu.VMEM((160,), jnp.int32),
                       pltpu.VMEM((ESUB, D), jnp.float32),
                       pltpu.VMEM((NBA + 1, D), jnp.float32),
                       pltpu.VMEM((NBA + 1, D), jnp.float32),
                       pltpu.VMEM(((NBA + 8) * 16,), jnp.float32),
                       pltpu.SemaphoreType.DMA],
        compiler_params=pltpu.CompilerParams(needs_layout_passes=False),
    )
    return f(dst, src, x)


def _stage_c_body(t_hbm, pi0_hbm, pi1_hbm, z_hbm, part_hbm,
                  idx0, idx1, urows, vrows, zbuf, pbuf, sem):
    c = lax.axis_index("c")
    s = lax.axis_index("s")
    wid = s * 2 + c
    base = wid * PW
    zero = jnp.zeros((16,), jnp.float32)

    def chunk(j, carry):
        s0, s1, q0, q1 = carry
        off = base + j * GC
        pltpu.sync_copy(pi0_hbm.at[pl.ds(off, GC)], idx0)
        pltpu.sync_copy(pi1_hbm.at[pl.ds(off, GC)], idx1)
        cp1 = pltpu.async_copy(t_hbm.at[idx0], urows, sem)
        cp2 = pltpu.async_copy(t_hbm.at[idx1], vrows, sem)
        cp1.wait()
        cp2.wait()

        def row(i, rc):
            rs0, rs1, rq0, rq1 = rc
            a0 = urows[i, pl.ds(0, 16)] + vrows[i, pl.ds(32, 16)]
            a1 = urows[i, pl.ds(16, 16)] + vrows[i, pl.ds(48, 16)]
            zbuf[i, pl.ds(0, 16)] = a0
            zbuf[i, pl.ds(16, 16)] = a1
            return (rs0 + a0, rs1 + a1, rq0 + a0 * a0, rq1 + a1 * a1)

        carry = lax.fori_loop(0, GC, row, (s0, s1, q0, q1))
        pltpu.sync_copy(zbuf, z_hbm.at[pl.ds(off, GC)])
        return carry

    s0, s1, q0, q1 = lax.fori_loop(0, NCH, chunk, (zero, zero, zero, zero))
    pbuf[pl.ds(0, 16)] = s0
    pbuf[pl.ds(16, 16)] = s1
    pbuf[pl.ds(32, 16)] = q0
    pbuf[pl.ds(48, 16)] = q1
    pltpu.sync_copy(pbuf, part_hbm.at[wid])


def _stage_c(t, pi0, pi1):
    f = pl.kernel(
        _stage_c_body,
        out_type=[jax.ShapeDtypeStruct((P, HID), jnp.float32),
                  jax.ShapeDtypeStruct((NW, 2 * HID), jnp.float32)],
        mesh=_sc_mesh(),
        scratch_types=[pltpu.VMEM((GC,), jnp.int32), pltpu.VMEM((GC,), jnp.int32),
                       pltpu.VMEM((GC, D), jnp.float32),
                       pltpu.VMEM((GC, D), jnp.float32),
                       pltpu.VMEM((GC, HID), jnp.float32),
                       pltpu.VMEM((2 * HID,), jnp.float32),
                       pltpu.SemaphoreType.DMA],
    )
    return f(t, pi0, pi1)


def kernel(x, edge_index, pairwise_indices, W_post, b_post, W1, b1, gamma, beta, W2, b2):
    src = edge_index[0]
    dst = edge_index[1]
    ssum, ssq, mx, mn, deg_flat = _stage_a(dst, src, x)
    deg = deg_flat.reshape(N, 16)[:, 0:1]
    delta = _delta(deg)
    t = _stage_b(x, ssum, ssq, mx, mn, deg, delta, W_post, b_post, W1)
    z, moments = _stage_c(t, pairwise_indices[0], pairwise_indices[1])
    return _stage_d(z, moments, gamma, beta, W2, b2)


# speedup vs baseline: 2.5179x; 1.2509x over previous
import functools
import jax
import jax.numpy as jnp
from jax import lax
from jax.experimental import pallas as pl
from jax.experimental.pallas import tpu as pltpu
from jax.experimental.pallas import tpu_sc as plsc

N = 10000
D = 128
HID = 32
P = 320000
NW = 32
PW = P // NW
GC = 80
NCH = PW // GC
FMAX = 3.0e38


def _sc_mesh():
    return plsc.VectorSubcoreMesh(core_axis_name="c", subcore_axis_name="s",
                                  num_cores=2, num_subcores=16)


def _stage_b_body(x_ref, ssum_ref, ssq_ref, mx_ref, mn_ref, deg_ref, delta_ref,
                  wp0_ref, wpa_ref, wpb_ref, wpc_ref, bpost_ref, w1p_ref,
                  t_ref):
    x = x_ref[...]
    deg = deg_ref[...]
    degc = jnp.maximum(deg, 1.0)
    mean = ssum_ref[...] / degc
    var = ssq_ref[...] / degc - mean * mean
    std = jnp.sqrt(jnp.maximum(var, 0.0) + 1e-5)
    pos = deg > 0.0
    mx = jnp.where(pos, mx_ref[...], 0.0)
    mn = jnp.where(pos, mn_ref[...], 0.0)
    A = jnp.concatenate([mean, mx, mn, std], axis=1)
    logd = jnp.log(deg + 1.0)
    delta = delta_ref[...]
    amp = logd / delta
    att = jnp.where(logd > 0.0, delta / jnp.maximum(logd, 1e-6), 1.0)
    f32 = jnp.float32
    h_pre = (jnp.dot(x, wp0_ref[...], preferred_element_type=f32)
             + jnp.dot(A, wpa_ref[...], preferred_element_type=f32)
             + jnp.dot(A * amp, wpb_ref[...], preferred_element_type=f32)
             + jnp.dot(A * att, wpc_ref[...], preferred_element_type=f32))
    h = jnp.maximum(h_pre + bpost_ref[...], 0.0) + x
    t_ref[...] = jnp.dot(h, w1p_ref[...], preferred_element_type=f32)


def _stage_b(x, ssum, ssq, mx, mn, deg, delta, W_post, b_post, W1):
    BN = 400
    grid = (N // BN,)
    full = lambda shape: pl.BlockSpec(shape, lambda i: (0, 0))
    blk = lambda cols: pl.BlockSpec((BN, cols), lambda i: (i, 0))
    Wp0 = W_post[:D]
    WpA = W_post[D:5 * D]
    WpB = W_post[5 * D:9 * D]
    WpC = W_post[9 * D:]
    W1p = jnp.concatenate([W1[:D], W1[D:], jnp.zeros((D, D - 2 * HID), jnp.float32)],
                          axis=1)
    return pl.pallas_call(
        _stage_b_body,
        grid=grid,
        in_specs=[blk(D), blk(D), blk(D), blk(D), blk(D), blk(1), full((1, 1)),
                  full((D, D)), full((4 * D, D)), full((4 * D, D)), full((4 * D, D)),
                  full((1, D)), full((D, D))],
        out_specs=blk(D),
        out_shape=jax.ShapeDtypeStruct((N, D), jnp.float32),
    )(x, ssum, ssq, mx, mn, deg, delta, Wp0, WpA, WpB, WpC,
      b_post.reshape(1, D), W1p)


def _delta_body(deg_ref, out_ref):
    out_ref[...] = (jnp.sum(jnp.log(deg_ref[...] + 1.0)) / N).reshape(1, 1)


def _delta(deg):
    return pl.pallas_call(
        _delta_body,
        in_specs=[pl.BlockSpec((N, 1), lambda: (0, 0))],
        out_specs=pl.BlockSpec((1, 1), lambda: (0, 0)),
        out_shape=jax.ShapeDtypeStruct((1, 1), jnp.float32),
    )(deg)


def _stage_d_body(z_ref, mom_ref, gamma_ref, beta_ref, w2_ref, b2_ref, out_ref, *, P):
    part = mom_ref[...]
    mu = jnp.sum(part[:, :HID], axis=0, keepdims=True) / P
    var = jnp.sum(part[:, HID:], axis=0, keepdims=True) / P - mu * mu
    inv = jax.lax.rsqrt(var + 1e-5)
    zn = (z_ref[...] - mu) * (inv * gamma_ref[...]) + beta_ref[...]
    zn = jnp.maximum(zn, 0.0)
    out_ref[...] = (jnp.dot(zn, w2_ref[...], preferred_element_type=jnp.float32)
                    + b2_ref[...])


def _stage_d(z, moments, gamma, beta, W2, b2):
    BP = 2000
    grid = (P // BP,)
    full = lambda shape: pl.BlockSpec(shape, lambda i: (0, 0))
    return pl.pallas_call(
        functools.partial(_stage_d_body, P=P),
        grid=grid,
        in_specs=[pl.BlockSpec((BP, HID), lambda i: (i, 0)), full((NW, 2 * HID)),
                  full((1, HID)), full((1, HID)), full((HID, 1)), full((1, 1))],
        out_specs=pl.BlockSpec((BP, 1), lambda i: (i, 0)),
        out_shape=jax.ShapeDtypeStruct((P, 1), jnp.float32),
    )(z, moments, gamma.reshape(1, HID), beta.reshape(1, HID), W2, b2.reshape(1, 1))


EE = 320000
ECH = 2560
ESC = 64
EGB = 128
ENSUB = ECH // ESC
ENCHK = EE // ECH
NBA = 320
ETRASH = 192


def _stage_a_body(dst_hbm, src_hbm, x_hbm,
                  ssum_hbm, ssq_hbm, mx_hbm, mn_hbm, deg_hbm,
                  dchunk_a, schunk_a, dchunk_b, schunk_b,
                  pend_dl, pend_src, rows, acc1, acc2, dega,
                  sem, csa, csb):
    c = lax.axis_index("c")
    s = lax.axis_index("s")
    r = s * 2 + c
    lo = r * NBA
    hi = lo + NBA
    i16 = lax.iota(jnp.int32, 16)
    e1 = (i16 == 0).astype(jnp.float32)
    zero_f = jnp.zeros((16,), jnp.float32)
    zero_i = jnp.zeros((16,), jnp.int32)
    cbufs = ((dchunk_a, schunk_a, csa), (dchunk_b, schunk_b, csb))

    def load_chunk(p, j):
        dc, sc_, csem = cbufs[p]
        pltpu.async_copy(dst_hbm.at[pl.ds(j * ECH, ECH)], dc, csem)
        pltpu.async_copy(src_hbm.at[pl.ds(j * ECH, ECH)], sc_, csem)

    def wait_chunk(p):
        dc, sc_, csem = cbufs[p]
        pltpu.make_async_copy(dst_hbm.at[pl.ds(0, ECH)], dc, csem).wait()
        pltpu.make_async_copy(src_hbm.at[pl.ds(0, ECH)], sc_, csem).wait()

    def init_phase(phase):
        v1 = zero_f if phase == 0 else jnp.full((16,), -FMAX, jnp.float32)
        v2 = zero_f if phase == 0 else jnp.full((16,), FMAX, jnp.float32)

        def row(i, carry):
            for cv in range(8):
                acc1[i, pl.ds(cv * 16, 16)] = v1
                acc2[i, pl.ds(cv * 16, 16)] = v2
            if phase == 0:
                dega[pl.ds(i * 16, 16)] = zero_f
            return carry

        lax.fori_loop(0, NBA + 1, row, 0)

    def accum_batch(phase):
        cp = pltpu.async_copy(x_hbm.at[pend_src.at[pl.ds(0, EGB)]], rows, sem)
        cp.wait()

        def acc_group(g, carry):
            dlv = pend_dl[pl.ds(g * 16, 16)]
            for t in range(16):
                dl = dlv[t]
                e = g * 16 + t
                for cv in range(8):
                    rv = rows[e, pl.ds(cv * 16, 16)]
                    a1 = acc1[dl, pl.ds(cv * 16, 16)]
                    a2 = acc2[dl, pl.ds(cv * 16, 16)]
                    if phase == 0:
                        acc1[dl, pl.ds(cv * 16, 16)] = a1 + rv
                        acc2[dl, pl.ds(cv * 16, 16)] = a2 + rv * rv
                    else:
                        acc1[dl, pl.ds(cv * 16, 16)] = jnp.maximum(a1, rv)
                        acc2[dl, pl.ds(cv * 16, 16)] = jnp.minimum(a2, rv)
                if phase == 0:
                    dv = dega[pl.ds(dl * 16, 16)]
                    dega[pl.ds(dl * 16, 16)] = dv + e1
            return carry

        lax.fori_loop(0, EGB // 16, acc_group, 0)

    def shift_residual():
        for w in range(ESC // 16):
            v_dl = pend_dl[pl.ds(EGB + w * 16, 16)]
            v_sr = pend_src[pl.ds(EGB + w * 16, 16)]
            pend_dl[pl.ds(w * 16, 16)] = v_dl
            pend_src[pl.ds(w * 16, 16)] = v_sr

    def run_phase(phase):
        init_phase(phase)

        def scan_chunk(p, np_vec):
            dc, sc_, _ = cbufs[p]

            def sub(si, np_vec):
                boff = si * ESC
                for k in range(ESC // 16):
                    dstv = dc[pl.ds(boff + k * 16, 16)]
                    srcv = sc_[pl.ds(boff + k * 16, 16)]
                    m = (dstv >= lo) & (dstv < hi)
                    rank = plsc.cumsum(m.astype(jnp.int32)) - 1
                    idx = jnp.where(m, np_vec + rank, ETRASH)
                    plsc.store_scatter(pend_dl, [idx], dstv - lo)
                    plsc.store_scatter(pend_src, [idx], srcv)
                    np_vec = np_vec + plsc.all_reduce_population_count(m)
                nps = lax.reduce_max(np_vec, axes=(0,))
                do = nps >= EGB

                @pl.when(do)
                def _():
                    accum_batch(phase)
                    shift_residual()

                return jnp.where(do, np_vec - EGB, np_vec)

            return lax.fori_loop(0, ENSUB, sub, np_vec)

        load_chunk(0, 0)

        def pairf(t, np_vec):
            load_chunk(1, 2 * t + 1)
            wait_chunk(0)
            np_vec = scan_chunk(0, np_vec)
            load_chunk(0, 2 * t + 2)
            wait_chunk(1)
            return scan_chunk(1, np_vec)

        np_vec = lax.fori_loop(0, ENCHK // 2, pairf, zero_i)
        wait_chunk(0)
        np_vec = scan_chunk(0, np_vec)
        for w in range(EGB // 16):
            pos = i16 + w * 16
            keep = pos < np_vec
            cur_dl = pend_dl[pl.ds(w * 16, 16)]
            cur_sr = pend_src[pl.ds(w * 16, 16)]
            pend_dl[pl.ds(w * 16, 16)] = jnp.where(keep, cur_dl, NBA)
            pend_src[pl.ds(w * 16, 16)] = jnp.where(keep, cur_sr, (r * 311) % N)
        accum_batch(phase)

        def write(nrows):
            row0 = r * NBA
            if phase == 0:
                pltpu.sync_copy(acc1.at[pl.ds(0, nrows)], ssum_hbm.at[pl.ds(row0, nrows)])
                pltpu.sync_copy(acc2.at[pl.ds(0, nrows)], ssq_hbm.at[pl.ds(row0, nrows)])
                pltpu.sync_copy(dega.at[pl.ds(0, nrows * 16)],
                                deg_hbm.at[pl.ds(row0 * 16, nrows * 16)])
            else:
                pltpu.sync_copy(acc1.at[pl.ds(0, nrows)], mx_hbm.at[pl.ds(row0, nrows)])
                pltpu.sync_copy(acc2.at[pl.ds(0, nrows)], mn_hbm.at[pl.ds(row0, nrows)])

        @pl.when(r < 31)
        def _():
            write(NBA)

        @pl.when(r == 31)
        def _():
            write(N - 31 * NBA)

    run_phase(0)
    run_phase(1)


def _stage_a(dst, src, x):
    f = pl.kernel(
        _stage_a_body,
        out_type=[jax.ShapeDtypeStruct((N, D), jnp.float32),
                  jax.ShapeDtypeStruct((N, D), jnp.float32),
                  jax.ShapeDtypeStruct((N, D), jnp.float32),
                  jax.ShapeDtypeStruct((N, D), jnp.float32),
                  jax.ShapeDtypeStruct((N * 16,), jnp.float32)],
        mesh=_sc_mesh(),
        scratch_types=[pltpu.VMEM((ECH,), jnp.int32), pltpu.VMEM((ECH,), jnp.int32),
                       pltpu.VMEM((ECH,), jnp.int32), pltpu.VMEM((ECH,), jnp.int32),
                       pltpu.VMEM((208,), jnp.int32), pltpu.VMEM((208,), jnp.int32),
                       pltpu.VMEM((EGB, D), jnp.float32),
                       pltpu.VMEM((NBA + 1, D), jnp.float32),
                       pltpu.VMEM((NBA + 1, D), jnp.float32),
                       pltpu.VMEM(((NBA + 8) * 16,), jnp.float32),
                       pltpu.SemaphoreType.DMA, pltpu.SemaphoreType.DMA,
                       pltpu.SemaphoreType.DMA],
        compiler_params=pltpu.CompilerParams(needs_layout_passes=False),
    )
    return f(dst, src, x)


def _stage_c_body(t_hbm, pi0_hbm, pi1_hbm, z_hbm, part_hbm,
                  idx0a, idx1a, ura, vra, zba,
                  idx0b, idx1b, urb, vrb, zbb,
                  pbuf, gsa, gsb, zsa, zsb):
    c = lax.axis_index("c")
    s = lax.axis_index("s")
    wid = s * 2 + c
    base = wid * PW
    zero = jnp.zeros((16,), jnp.float32)
    bufs = ((idx0a, idx1a, ura, vra, zba, gsa, zsa),
            (idx0b, idx1b, urb, vrb, zbb, gsb, zsb))

    def start(p, off):
        idx0, idx1, urows, vrows, _, gsem, _ = bufs[p]
        pltpu.sync_copy(pi0_hbm.at[pl.ds(off, GC)], idx0)
        pltpu.sync_copy(pi1_hbm.at[pl.ds(off, GC)], idx1)
        pltpu.async_copy(t_hbm.at[idx0], urows, gsem)
        pltpu.async_copy(t_hbm.at[idx1], vrows, gsem)

    def wait_g(p):
        idx0, idx1, urows, vrows, _, gsem, _ = bufs[p]
        pltpu.make_async_copy(t_hbm.at[idx0], urows, gsem).wait()
        pltpu.make_async_copy(t_hbm.at[idx1], vrows, gsem).wait()

    def wait_z(p):
        zbuf, zsem = bufs[p][4], bufs[p][6]
        pltpu.make_async_copy(zbuf, z_hbm.at[pl.ds(0, GC)], zsem).wait()

    def compute(p, off, carry):
        _, _, urows, vrows, zbuf, _, zsem = bufs[p]

        def row(i, rc):
            rs0, rs1, rq0, rq1 = rc
            a0 = urows[i, pl.ds(0, 16)] + vrows[i, pl.ds(32, 16)]
            a1 = urows[i, pl.ds(16, 16)] + vrows[i, pl.ds(48, 16)]
            zbuf[i, pl.ds(0, 16)] = a0
            zbuf[i, pl.ds(16, 16)] = a1
            return (rs0 + a0, rs1 + a1, rq0 + a0 * a0, rq1 + a1 * a1)

        carry = lax.fori_loop(0, GC, row, carry)
        pltpu.async_copy(zbuf, z_hbm.at[pl.ds(off, GC)], zsem)
        return carry

    TPAIR = NCH // 2
    start(0, base)

    def pair(t, carry):
        off0 = base + (2 * t) * GC
        start(1, off0 + GC)
        wait_g(0)

        @pl.when(t > 0)
        def _():
            wait_z(0)

        carry = compute(0, off0, carry)
        start(0, off0 + 2 * GC)
        wait_g(1)

        @pl.when(t > 0)
        def _():
            wait_z(1)

        return compute(1, off0 + GC, carry)

    carry = lax.fori_loop(0, TPAIR, pair, (zero, zero, zero, zero))
    wait_g(0)
    wait_z(0)
    s0, s1, q0, q1 = compute(0, base + (NCH - 1) * GC, carry)
    wait_z(0)
    wait_z(1)
    pbuf[pl.ds(0, 16)] = s0
    pbuf[pl.ds(16, 16)] = s1
    pbuf[pl.ds(32, 16)] = q0
    pbuf[pl.ds(48, 16)] = q1
    pltpu.sync_copy(pbuf, part_hbm.at[wid])


def _stage_c(t, pi0, pi1):
    dbl = lambda: [pltpu.VMEM((GC,), jnp.int32), pltpu.VMEM((GC,), jnp.int32),
                   pltpu.VMEM((GC, D), jnp.float32), pltpu.VMEM((GC, D), jnp.float32),
                   pltpu.VMEM((GC, HID), jnp.float32)]
    f = pl.kernel(
        _stage_c_body,
        out_type=[jax.ShapeDtypeStruct((P, HID), jnp.float32),
                  jax.ShapeDtypeStruct((NW, 2 * HID), jnp.float32)],
        mesh=_sc_mesh(),
        scratch_types=dbl() + dbl() + [pltpu.VMEM((2 * HID,), jnp.float32),
                                       pltpu.SemaphoreType.DMA,
                                       pltpu.SemaphoreType.DMA,
                                       pltpu.SemaphoreType.DMA,
                                       pltpu.SemaphoreType.DMA],
        compiler_params=pltpu.CompilerParams(needs_layout_passes=False),
    )
    return f(t, pi0, pi1)


def kernel(x, edge_index, pairwise_indices, W_post, b_post, W1, b1, gamma, beta, W2, b2):
    src = edge_index[0]
    dst = edge_index[1]
    ssum, ssq, mx, mn, deg_flat = _stage_a(dst, src, x)
    deg = deg_flat.reshape(N, 16)[:, 0:1]
    delta = _delta(deg)
    t = _stage_b(x, ssum, ssq, mx, mn, deg, delta, W_post, b_post, W1)
    z, moments = _stage_c(t, pairwise_indices[0], pairwise_indices[1])
    return _stage_d(z, moments, gamma, beta, W2, b2)


# speedup vs baseline: 2.8841x; 1.1454x over previous
import functools
import jax
import jax.numpy as jnp
from jax import lax
from jax.experimental import pallas as pl
from jax.experimental.pallas import tpu as pltpu
from jax.experimental.pallas import tpu_sc as plsc

N = 10000
D = 128
HID = 32
P = 320000
NW = 32
PW = P // NW
GC = 80
NCH = PW // GC
FMAX = 3.0e38


def _sc_mesh():
    return plsc.VectorSubcoreMesh(core_axis_name="c", subcore_axis_name="s",
                                  num_cores=2, num_subcores=16)


def _stage_b_body(x_ref, ssum_ref, ssq_ref, mx_ref, mn_ref, deg_ref, delta_ref,
                  wp0_ref, wpa_ref, wpb_ref, wpc_ref, bpost_ref, w1p_ref,
                  t_ref):
    x = x_ref[...]
    deg = deg_ref[...]
    degc = jnp.maximum(deg, 1.0)
    mean = ssum_ref[...] / degc
    var = ssq_ref[...] / degc - mean * mean
    std = jnp.sqrt(jnp.maximum(var, 0.0) + 1e-5)
    pos = deg > 0.0
    mx = jnp.where(pos, mx_ref[...], 0.0)
    mn = jnp.where(pos, mn_ref[...], 0.0)
    A = jnp.concatenate([mean, mx, mn, std], axis=1)
    logd = jnp.log(deg + 1.0)
    delta = delta_ref[...]
    amp = logd / delta
    att = jnp.where(logd > 0.0, delta / jnp.maximum(logd, 1e-6), 1.0)
    f32 = jnp.float32
    h_pre = (jnp.dot(x, wp0_ref[...], preferred_element_type=f32)
             + jnp.dot(A, wpa_ref[...], preferred_element_type=f32)
             + jnp.dot(A * amp, wpb_ref[...], preferred_element_type=f32)
             + jnp.dot(A * att, wpc_ref[...], preferred_element_type=f32))
    h = jnp.maximum(h_pre + bpost_ref[...], 0.0) + x
    t_ref[...] = jnp.dot(h, w1p_ref[...], preferred_element_type=f32)


def _stage_b(x, ssum, ssq, mx, mn, deg, delta, W_post, b_post, W1):
    BN = 400
    grid = (N // BN,)
    full = lambda shape: pl.BlockSpec(shape, lambda i: (0, 0))
    blk = lambda cols: pl.BlockSpec((BN, cols), lambda i: (i, 0))
    Wp0 = W_post[:D]
    WpA = W_post[D:5 * D]
    WpB = W_post[5 * D:9 * D]
    WpC = W_post[9 * D:]
    W1p = jnp.concatenate([W1[:D], W1[D:], jnp.zeros((D, D - 2 * HID), jnp.float32)],
                          axis=1)
    return pl.pallas_call(
        _stage_b_body,
        grid=grid,
        in_specs=[blk(D), blk(D), blk(D), blk(D), blk(D), blk(1), full((1, 1)),
                  full((D, D)), full((4 * D, D)), full((4 * D, D)), full((4 * D, D)),
                  full((1, D)), full((D, D))],
        out_specs=blk(D),
        out_shape=jax.ShapeDtypeStruct((N, D), jnp.float32),
    )(x, ssum, ssq, mx, mn, deg, delta, Wp0, WpA, WpB, WpC,
      b_post.reshape(1, D), W1p)


def _delta_body(deg_ref, out_ref):
    out_ref[...] = (jnp.sum(jnp.log(deg_ref[...] + 1.0)) / N).reshape(1, 1)


def _delta(deg):
    return pl.pallas_call(
        _delta_body,
        in_specs=[pl.BlockSpec((N, 1), lambda: (0, 0))],
        out_specs=pl.BlockSpec((1, 1), lambda: (0, 0)),
        out_shape=jax.ShapeDtypeStruct((1, 1), jnp.float32),
    )(deg)


def _stage_d_body(z_ref, mom_ref, gamma_ref, beta_ref, w2_ref, b2_ref, out_ref, *, P):
    part = mom_ref[...]
    mu = jnp.sum(part[:, :HID], axis=0, keepdims=True) / P
    var = jnp.sum(part[:, HID:], axis=0, keepdims=True) / P - mu * mu
    inv = jax.lax.rsqrt(var + 1e-5)
    zn = (z_ref[...] - mu) * (inv * gamma_ref[...]) + beta_ref[...]
    zn = jnp.maximum(zn, 0.0)
    out_ref[...] = (jnp.dot(zn, w2_ref[...], preferred_element_type=jnp.float32)
                    + b2_ref[...])


def _stage_d(z, moments, gamma, beta, W2, b2):
    BP = 2000
    grid = (P // BP,)
    full = lambda shape: pl.BlockSpec(shape, lambda i: (0, 0))
    return pl.pallas_call(
        functools.partial(_stage_d_body, P=P),
        grid=grid,
        in_specs=[pl.BlockSpec((BP, HID), lambda i: (i, 0)), full((NW, 2 * HID)),
                  full((1, HID)), full((1, HID)), full((HID, 1)), full((1, 1))],
        out_specs=pl.BlockSpec((BP, 1), lambda i: (i, 0)),
        out_shape=jax.ShapeDtypeStruct((P, 1), jnp.float32),
    )(z, moments, gamma.reshape(1, HID), beta.reshape(1, HID), W2, b2.reshape(1, 1))


EE = 320000
ECH = 2560
ESC = 64
EGB = 128
ENSUB = ECH // ESC
ENCHK = EE // ECH
NBA = 320
ETRASH = 192


def _stage_a_body(dst_hbm, src_hbm, x_hbm,
                  ssum_hbm, ssq_hbm, mx_hbm, mn_hbm, deg_hbm, elist_hbm,
                  dchunk_a, schunk_a, dchunk_b, schunk_b,
                  pend_dl, pend_src, rows, acc1, acc2, dega, sbuf,
                  sem, csa, csb, swsem):
    c = lax.axis_index("c")
    s = lax.axis_index("s")
    r = s * 2 + c
    lo = r * NBA
    hi = lo + NBA
    i16 = lax.iota(jnp.int32, 16)
    e1 = (i16 == 0).astype(jnp.float32)
    zero_f = jnp.zeros((16,), jnp.float32)
    zero_i = jnp.zeros((16,), jnp.int32)
    cbufs = ((dchunk_a, schunk_a, csa), (dchunk_b, schunk_b, csb))

    def load_chunk(p, j):
        dc, sc_, csem = cbufs[p]
        pltpu.async_copy(dst_hbm.at[pl.ds(j * ECH, ECH)], dc, csem)
        pltpu.async_copy(src_hbm.at[pl.ds(j * ECH, ECH)], sc_, csem)

    def wait_chunk(p):
        dc, sc_, csem = cbufs[p]
        pltpu.make_async_copy(dst_hbm.at[pl.ds(0, ECH)], dc, csem).wait()
        pltpu.make_async_copy(src_hbm.at[pl.ds(0, ECH)], sc_, csem).wait()

    def init_phase(phase):
        v1 = zero_f if phase == 0 else jnp.full((16,), -FMAX, jnp.float32)
        v2 = zero_f if phase == 0 else jnp.full((16,), FMAX, jnp.float32)

        def row(i, carry):
            for cv in range(8):
                acc1[i, pl.ds(cv * 16, 16)] = v1
                acc2[i, pl.ds(cv * 16, 16)] = v2
            if phase == 0:
                dega[pl.ds(i * 16, 16)] = zero_f
            return carry

        lax.fori_loop(0, NBA + 1, row, 0)

    def accum_batch(phase):
        cp = pltpu.async_copy(x_hbm.at[pend_src.at[pl.ds(0, EGB)]], rows, sem)
        cp.wait()

        def acc_group(g, carry):
            dlv = pend_dl[pl.ds(g * 16, 16)]
            for t in range(16):
                dl = dlv[t]
                e = g * 16 + t
                for cv in range(8):
                    rv = rows[e, pl.ds(cv * 16, 16)]
                    a1 = acc1[dl, pl.ds(cv * 16, 16)]
                    a2 = acc2[dl, pl.ds(cv * 16, 16)]
                    if phase == 0:
                        acc1[dl, pl.ds(cv * 16, 16)] = a1 + rv
                        acc2[dl, pl.ds(cv * 16, 16)] = a2 + rv * rv
                    else:
                        acc1[dl, pl.ds(cv * 16, 16)] = jnp.maximum(a1, rv)
                        acc2[dl, pl.ds(cv * 16, 16)] = jnp.minimum(a2, rv)
                if phase == 0:
                    dv = dega[pl.ds(dl * 16, 16)]
                    dega[pl.ds(dl * 16, 16)] = dv + e1
            return carry

        lax.fori_loop(0, EGB // 16, acc_group, 0)

    def shift_residual():
        for w in range(ESC // 16):
            v_dl = pend_dl[pl.ds(EGB + w * 16, 16)]
            v_sr = pend_src[pl.ds(EGB + w * 16, 16)]
            pend_dl[pl.ds(w * 16, 16)] = v_dl
            pend_src[pl.ds(w * 16, 16)] = v_sr

    def pack_batch(woff):
        @pl.when(woff > 0)
        def _():
            pltpu.make_async_copy(sbuf, elist_hbm.at[pl.ds(0, EGB)], swsem).wait()
        for k in range(EGB // 16):
            dlv = pend_dl[pl.ds(k * 16, 16)]
            srv = pend_src[pl.ds(k * 16, 16)]
            sbuf[pl.ds(k * 16, 16)] = dlv * 16384 + srv
        pltpu.async_copy(sbuf, elist_hbm.at[pl.ds(r * EE + pl.multiple_of(woff, EGB), EGB)], swsem)

    def run_phase(phase):
        init_phase(phase)

        def scan_chunk(p, carry):
            dc, sc_, _ = cbufs[p]

            def sub(si, carry):
                np_vec, woff = carry
                boff = si * ESC
                for k in range(ESC // 16):
                    dstv = dc[pl.ds(boff + k * 16, 16)]
                    srcv = sc_[pl.ds(boff + k * 16, 16)]
                    m = (dstv >= lo) & (dstv < hi)
                    rank = plsc.cumsum(m.astype(jnp.int32)) - 1
                    idx = jnp.where(m, np_vec + rank, ETRASH)
                    plsc.store_scatter(pend_dl, [idx], dstv - lo)
                    plsc.store_scatter(pend_src, [idx], srcv)
                    np_vec = np_vec + plsc.all_reduce_population_count(m)
                nps = lax.reduce_max(np_vec, axes=(0,))
                do = nps >= EGB

                @pl.when(do)
                def _():
                    accum_batch(phase)
                    pack_batch(woff)
                    shift_residual()

                return (jnp.where(do, np_vec - EGB, np_vec),
                        jnp.where(do, woff + EGB, woff))

            return lax.fori_loop(0, ENSUB, sub, carry)

        load_chunk(0, 0)

        def pairf(t, carry):
            load_chunk(1, 2 * t + 1)
            wait_chunk(0)
            carry = scan_chunk(0, carry)
            load_chunk(0, 2 * t + 2)
            wait_chunk(1)
            return scan_chunk(1, carry)

        np_vec, woff = lax.fori_loop(0, ENCHK // 2, pairf, (zero_i, jnp.int32(0)))
        wait_chunk(0)
        np_vec, woff = scan_chunk(0, (np_vec, woff))
        for w in range(EGB // 16):
            pos = i16 + w * 16
            keep = pos < np_vec
            cur_dl = pend_dl[pl.ds(w * 16, 16)]
            cur_sr = pend_src[pl.ds(w * 16, 16)]
            pend_dl[pl.ds(w * 16, 16)] = jnp.where(keep, cur_dl, NBA)
            pend_src[pl.ds(w * 16, 16)] = jnp.where(keep, cur_sr, (r * 311) % N)
        accum_batch(phase)
        pack_batch(woff)
        pltpu.make_async_copy(sbuf, elist_hbm.at[pl.ds(0, EGB)], swsem).wait()
        return woff // EGB + 1

    def run_replay():
        init_phase(1)

        def batchf(b, carry):
            pltpu.sync_copy(elist_hbm.at[pl.ds(r * EE + b * EGB, EGB)], sbuf)
            for k in range(EGB // 16):
                v = sbuf[pl.ds(k * 16, 16)]
                pend_dl[pl.ds(k * 16, 16)] = jnp.right_shift(v, 14)
                pend_src[pl.ds(k * 16, 16)] = jnp.bitwise_and(v, 16383)
            accum_batch(1)
            return carry

        lax.fori_loop(0, nb, batchf, 0)

    def write_out(phase, nrows):
        row0 = r * NBA
        if phase == 0:
            pltpu.sync_copy(acc1.at[pl.ds(0, nrows)], ssum_hbm.at[pl.ds(row0, nrows)])
            pltpu.sync_copy(acc2.at[pl.ds(0, nrows)], ssq_hbm.at[pl.ds(row0, nrows)])
            pltpu.sync_copy(dega.at[pl.ds(0, nrows * 16)],
                            deg_hbm.at[pl.ds(row0 * 16, nrows * 16)])
        else:
            pltpu.sync_copy(acc1.at[pl.ds(0, nrows)], mx_hbm.at[pl.ds(row0, nrows)])
            pltpu.sync_copy(acc2.at[pl.ds(0, nrows)], mn_hbm.at[pl.ds(row0, nrows)])

    def finish(phase):
        @pl.when(r < 31)
        def _():
            write_out(phase, NBA)

        @pl.when(r == 31)
        def _():
            write_out(phase, N - 31 * NBA)

    nb = run_phase(0)
    finish(0)
    run_replay()
    finish(1)


def _stage_a(dst, src, x):
    f = pl.kernel(
        _stage_a_body,
        out_type=[jax.ShapeDtypeStruct((N, D), jnp.float32),
                  jax.ShapeDtypeStruct((N, D), jnp.float32),
                  jax.ShapeDtypeStruct((N, D), jnp.float32),
                  jax.ShapeDtypeStruct((N, D), jnp.float32),
                  jax.ShapeDtypeStruct((N * 16,), jnp.float32),
                  jax.ShapeDtypeStruct((NW * EE,), jnp.int32)],
        mesh=_sc_mesh(),
        scratch_types=[pltpu.VMEM((ECH,), jnp.int32), pltpu.VMEM((ECH,), jnp.int32),
                       pltpu.VMEM((ECH,), jnp.int32), pltpu.VMEM((ECH,), jnp.int32),
                       pltpu.VMEM((208,), jnp.int32), pltpu.VMEM((208,), jnp.int32),
                       pltpu.VMEM((EGB, D), jnp.float32),
                       pltpu.VMEM((NBA + 1, D), jnp.float32),
                       pltpu.VMEM((NBA + 1, D), jnp.float32),
                       pltpu.VMEM(((NBA + 8) * 16,), jnp.float32),
                       pltpu.VMEM((EGB,), jnp.int32),
                       pltpu.SemaphoreType.DMA, pltpu.SemaphoreType.DMA,
                       pltpu.SemaphoreType.DMA, pltpu.SemaphoreType.DMA],
        compiler_params=pltpu.CompilerParams(needs_layout_passes=False),
    )
    return f(dst, src, x)[:5]


def _stage_c_body(t_hbm, pi0_hbm, pi1_hbm, z_hbm, part_hbm,
                  idx0a, idx1a, ura, vra, zba,
                  idx0b, idx1b, urb, vrb, zbb,
                  pbuf, gsa, gsb, zsa, zsb):
    c = lax.axis_index("c")
    s = lax.axis_index("s")
    wid = s * 2 + c
    base = wid * PW
    zero = jnp.zeros((16,), jnp.float32)
    bufs = ((idx0a, idx1a, ura, vra, zba, gsa, zsa),
            (idx0b, idx1b, urb, vrb, zbb, gsb, zsb))

    def start(p, off):
        idx0, idx1, urows, vrows, _, gsem, _ = bufs[p]
        pltpu.sync_copy(pi0_hbm.at[pl.ds(off, GC)], idx0)
        pltpu.sync_copy(pi1_hbm.at[pl.ds(off, GC)], idx1)
        pltpu.async_copy(t_hbm.at[idx0], urows, gsem)
        pltpu.async_copy(t_hbm.at[idx1], vrows, gsem)

    def wait_g(p):
        idx0, idx1, urows, vrows, _, gsem, _ = bufs[p]
        pltpu.make_async_copy(t_hbm.at[idx0], urows, gsem).wait()
        pltpu.make_async_copy(t_hbm.at[idx1], vrows, gsem).wait()

    def wait_z(p):
        zbuf, zsem = bufs[p][4], bufs[p][6]
        pltpu.make_async_copy(zbuf, z_hbm.at[pl.ds(0, GC)], zsem).wait()

    def compute(p, off, carry):
        _, _, urows, vrows, zbuf, _, zsem = bufs[p]

        def row(i, rc):
            rs0, rs1, rq0, rq1 = rc
            a0 = urows[i, pl.ds(0, 16)] + vrows[i, pl.ds(32, 16)]
            a1 = urows[i, pl.ds(16, 16)] + vrows[i, pl.ds(48, 16)]
            zbuf[i, pl.ds(0, 16)] = a0
            zbuf[i, pl.ds(16, 16)] = a1
            return (rs0 + a0, rs1 + a1, rq0 + a0 * a0, rq1 + a1 * a1)

        carry = lax.fori_loop(0, GC, row, carry)
        pltpu.async_copy(zbuf, z_hbm.at[pl.ds(off, GC)], zsem)
        return carry

    TPAIR = NCH // 2
    start(0, base)

    def pair(t, carry):
        off0 = base + (2 * t) * GC
        start(1, off0 + GC)
        wait_g(0)

        @pl.when(t > 0)
        def _():
            wait_z(0)

        carry = compute(0, off0, carry)
        start(0, off0 + 2 * GC)
        wait_g(1)

        @pl.when(t > 0)
        def _():
            wait_z(1)

        return compute(1, off0 + GC, carry)

    carry = lax.fori_loop(0, TPAIR, pair, (zero, zero, zero, zero))
    wait_g(0)
    wait_z(0)
    s0, s1, q0, q1 = compute(0, base + (NCH - 1) * GC, carry)
    wait_z(0)
    wait_z(1)
    pbuf[pl.ds(0, 16)] = s0
    pbuf[pl.ds(16, 16)] = s1
    pbuf[pl.ds(32, 16)] = q0
    pbuf[pl.ds(48, 16)] = q1
    pltpu.sync_copy(pbuf, part_hbm.at[wid])


def _stage_c(t, pi0, pi1):
    dbl = lambda: [pltpu.VMEM((GC,), jnp.int32), pltpu.VMEM((GC,), jnp.int32),
                   pltpu.VMEM((GC, D), jnp.float32), pltpu.VMEM((GC, D), jnp.float32),
                   pltpu.VMEM((GC, HID), jnp.float32)]
    f = pl.kernel(
        _stage_c_body,
        out_type=[jax.ShapeDtypeStruct((P, HID), jnp.float32),
                  jax.ShapeDtypeStruct((NW, 2 * HID), jnp.float32)],
        mesh=_sc_mesh(),
        scratch_types=dbl() + dbl() + [pltpu.VMEM((2 * HID,), jnp.float32),
                                       pltpu.SemaphoreType.DMA,
                                       pltpu.SemaphoreType.DMA,
                                       pltpu.SemaphoreType.DMA,
                                       pltpu.SemaphoreType.DMA],
        compiler_params=pltpu.CompilerParams(needs_layout_passes=False),
    )
    return f(t, pi0, pi1)


def kernel(x, edge_index, pairwise_indices, W_post, b_post, W1, b1, gamma, beta, W2, b2):
    src = edge_index[0]
    dst = edge_index[1]
    ssum, ssq, mx, mn, deg_flat = _stage_a(dst, src, x)
    deg = deg_flat.reshape(N, 16)[:, 0:1]
    delta = _delta(deg)
    t = _stage_b(x, ssum, ssq, mx, mn, deg, delta, W_post, b_post, W1)
    z, moments = _stage_c(t, pairwise_indices[0], pairwise_indices[1])
    return _stage_d(z, moments, gamma, beta, W2, b2)


# speedup vs baseline: 2.9031x; 1.0066x over previous
import functools
import jax
import jax.numpy as jnp
from jax import lax
from jax.experimental import pallas as pl
from jax.experimental.pallas import tpu as pltpu
from jax.experimental.pallas import tpu_sc as plsc

N = 10000
D = 128
HID = 32
P = 320000
NW = 32
PW = P // NW
GC = 80
NCH = PW // GC
FMAX = 3.0e38


def _sc_mesh():
    return plsc.VectorSubcoreMesh(core_axis_name="c", subcore_axis_name="s",
                                  num_cores=2, num_subcores=16)


def _stage_b_body(x_ref, ssum_ref, ssq_ref, mx_ref, mn_ref, deg_ref, delta_ref,
                  wp0_ref, wpa_ref, wpb_ref, wpc_ref, bpost_ref, w1p_ref,
                  t_ref):
    x = x_ref[...]
    deg = deg_ref[...]
    degc = jnp.maximum(deg, 1.0)
    mean = ssum_ref[...] / degc
    var = ssq_ref[...] / degc - mean * mean
    std = jnp.sqrt(jnp.maximum(var, 0.0) + 1e-5)
    pos = deg > 0.0
    mx = jnp.where(pos, mx_ref[...], 0.0)
    mn = jnp.where(pos, mn_ref[...], 0.0)
    A = jnp.concatenate([mean, mx, mn, std], axis=1)
    logd = jnp.log(deg + 1.0)
    delta = delta_ref[...]
    amp = logd / delta
    att = jnp.where(logd > 0.0, delta / jnp.maximum(logd, 1e-6), 1.0)
    f32 = jnp.float32
    h_pre = (jnp.dot(x, wp0_ref[...], preferred_element_type=f32)
             + jnp.dot(A, wpa_ref[...], preferred_element_type=f32)
             + jnp.dot(A * amp, wpb_ref[...], preferred_element_type=f32)
             + jnp.dot(A * att, wpc_ref[...], preferred_element_type=f32))
    h = jnp.maximum(h_pre + bpost_ref[...], 0.0) + x
    t_ref[...] = jnp.dot(h, w1p_ref[...], preferred_element_type=f32)


def _stage_b(x, ssum, ssq, mx, mn, deg, delta, W_post, b_post, W1):
    BN = 400
    grid = (N // BN,)
    full = lambda shape: pl.BlockSpec(shape, lambda i: (0, 0))
    blk = lambda cols: pl.BlockSpec((BN, cols), lambda i: (i, 0))
    Wp0 = W_post[:D]
    WpA = W_post[D:5 * D]
    WpB = W_post[5 * D:9 * D]
    WpC = W_post[9 * D:]
    W1p = jnp.concatenate([W1[:D], W1[D:], jnp.zeros((D, D - 2 * HID), jnp.float32)],
                          axis=1)
    return pl.pallas_call(
        _stage_b_body,
        grid=grid,
        in_specs=[blk(D), blk(D), blk(D), blk(D), blk(D), blk(1), full((1, 1)),
                  full((D, D)), full((4 * D, D)), full((4 * D, D)), full((4 * D, D)),
                  full((1, D)), full((D, D))],
        out_specs=blk(D),
        out_shape=jax.ShapeDtypeStruct((N, D), jnp.float32),
    )(x, ssum, ssq, mx, mn, deg, delta, Wp0, WpA, WpB, WpC,
      b_post.reshape(1, D), W1p)


def _delta_body(deg_ref, out_ref):
    out_ref[...] = (jnp.sum(jnp.log(deg_ref[...] + 1.0)) / N).reshape(1, 1)


def _delta(deg):
    return pl.pallas_call(
        _delta_body,
        in_specs=[pl.BlockSpec((N, 1), lambda: (0, 0))],
        out_specs=pl.BlockSpec((1, 1), lambda: (0, 0)),
        out_shape=jax.ShapeDtypeStruct((1, 1), jnp.float32),
    )(deg)


def _stage_d_body(z_ref, mom_ref, gamma_ref, beta_ref, w2_ref, b2_ref, out_ref, *, P):
    part = mom_ref[...]
    mu = jnp.sum(part[:, :HID], axis=0, keepdims=True) / P
    var = jnp.sum(part[:, HID:], axis=0, keepdims=True) / P - mu * mu
    inv = jax.lax.rsqrt(var + 1e-5)
    zn = (z_ref[...] - mu) * (inv * gamma_ref[...]) + beta_ref[...]
    zn = jnp.maximum(zn, 0.0)
    out_ref[...] = (jnp.dot(zn, w2_ref[...], preferred_element_type=jnp.float32)
                    + b2_ref[...])


def _stage_d(z, moments, gamma, beta, W2, b2):
    BP = 2000
    grid = (P // BP,)
    full = lambda shape: pl.BlockSpec(shape, lambda i: (0, 0))
    return pl.pallas_call(
        functools.partial(_stage_d_body, P=P),
        grid=grid,
        in_specs=[pl.BlockSpec((BP, HID), lambda i: (i, 0)), full((NW, 2 * HID)),
                  full((1, HID)), full((1, HID)), full((HID, 1)), full((1, 1))],
        out_specs=pl.BlockSpec((BP, 1), lambda i: (i, 0)),
        out_shape=jax.ShapeDtypeStruct((P, 1), jnp.float32),
    )(z, moments, gamma.reshape(1, HID), beta.reshape(1, HID), W2, b2.reshape(1, 1))


EE = 320000
ECH = 1280
ESC = 64
EGB = 128
ENSUB = ECH // ESC
ENCHK = EE // ECH
NBA = 320
ETRASH = 192


def _stage_a_body(dst_hbm, src_hbm, x_hbm,
                  ssum_hbm, ssq_hbm, mx_hbm, mn_hbm, deg_hbm, elist_hbm,
                  dchunk_a, schunk_a, dchunk_b, schunk_b,
                  pend_dl, pend_src, rows, acc1, acc2, dega, sbuf,
                  sem, csa, csb, swsem):
    c = lax.axis_index("c")
    s = lax.axis_index("s")
    r = s * 2 + c
    lo = r * NBA
    hi = lo + NBA
    i16 = lax.iota(jnp.int32, 16)
    e1 = (i16 == 0).astype(jnp.float32)
    zero_f = jnp.zeros((16,), jnp.float32)
    zero_i = jnp.zeros((16,), jnp.int32)
    cbufs = ((dchunk_a, schunk_a, csa), (dchunk_b, schunk_b, csb))

    def load_chunk(p, j):
        dc, sc_, csem = cbufs[p]
        pltpu.async_copy(dst_hbm.at[pl.ds(j * ECH, ECH)], dc, csem)
        pltpu.async_copy(src_hbm.at[pl.ds(j * ECH, ECH)], sc_, csem)

    def wait_chunk(p):
        dc, sc_, csem = cbufs[p]
        pltpu.make_async_copy(dst_hbm.at[pl.ds(0, ECH)], dc, csem).wait()
        pltpu.make_async_copy(src_hbm.at[pl.ds(0, ECH)], sc_, csem).wait()

    def init_phase(phase):
        v1 = zero_f if phase == 0 else jnp.full((16,), -FMAX, jnp.float32)
        v2 = zero_f if phase == 0 else jnp.full((16,), FMAX, jnp.float32)

        def row(i, carry):
            for cv in range(8):
                acc1[i, pl.ds(cv * 16, 16)] = v1
                acc2[i, pl.ds(cv * 16, 16)] = v2
            if phase == 0:
                dega[pl.ds(i * 16, 16)] = zero_f
            return carry

        lax.fori_loop(0, NBA + 1, row, 0)

    def accum_batch(phase, ngroups=EGB // 16):
        for hh in range(ngroups * 16 // EGB):
            pltpu.async_copy(x_hbm.at[pend_src.at[pl.ds(hh * EGB, EGB)]],
                             rows.at[pl.ds(hh * EGB, EGB)], sem)
        for hh in range(ngroups * 16 // EGB):
            pltpu.make_async_copy(x_hbm.at[pend_src.at[pl.ds(0, EGB)]],
                                  rows.at[pl.ds(0, EGB)], sem).wait()

        def acc_group(g, carry):
            dlv = pend_dl[pl.ds(g * 16, 16)]
            for t in range(16):
                dl = dlv[t]
                e = g * 16 + t
                for cv in range(8):
                    rv = rows[e, pl.ds(cv * 16, 16)]
                    a1 = acc1[dl, pl.ds(cv * 16, 16)]
                    a2 = acc2[dl, pl.ds(cv * 16, 16)]
                    if phase == 0:
                        acc1[dl, pl.ds(cv * 16, 16)] = a1 + rv
                        acc2[dl, pl.ds(cv * 16, 16)] = a2 + rv * rv
                    else:
                        acc1[dl, pl.ds(cv * 16, 16)] = jnp.maximum(a1, rv)
                        acc2[dl, pl.ds(cv * 16, 16)] = jnp.minimum(a2, rv)
                if phase == 0:
                    dv = dega[pl.ds(dl * 16, 16)]
                    dega[pl.ds(dl * 16, 16)] = dv + e1
            return carry

        lax.fori_loop(0, ngroups, acc_group, 0)

    def shift_residual():
        for w in range(ESC // 16):
            v_dl = pend_dl[pl.ds(EGB + w * 16, 16)]
            v_sr = pend_src[pl.ds(EGB + w * 16, 16)]
            pend_dl[pl.ds(w * 16, 16)] = v_dl
            pend_src[pl.ds(w * 16, 16)] = v_sr

    def pack_batch(woff):
        @pl.when(woff > 0)
        def _():
            pltpu.make_async_copy(sbuf.at[pl.ds(0, EGB)],
                                  elist_hbm.at[pl.ds(0, EGB)], swsem).wait()
        for k in range(EGB // 16):
            dlv = pend_dl[pl.ds(k * 16, 16)]
            srv = pend_src[pl.ds(k * 16, 16)]
            sbuf[pl.ds(k * 16, 16)] = dlv * 16384 + srv
        pltpu.async_copy(sbuf.at[pl.ds(0, EGB)],
                         elist_hbm.at[pl.ds(r * EE + pl.multiple_of(woff, EGB), EGB)],
                         swsem)

    def run_phase(phase):
        init_phase(phase)

        def scan_chunk(p, carry):
            dc, sc_, _ = cbufs[p]

            def sub(si, carry):
                np_vec, woff = carry
                boff = si * ESC
                for k in range(ESC // 16):
                    dstv = dc[pl.ds(boff + k * 16, 16)]
                    srcv = sc_[pl.ds(boff + k * 16, 16)]
                    m = (dstv >= lo) & (dstv < hi)
                    rank = plsc.cumsum(m.astype(jnp.int32)) - 1
                    idx = jnp.where(m, np_vec + rank, ETRASH)
                    plsc.store_scatter(pend_dl, [idx], dstv - lo)
                    plsc.store_scatter(pend_src, [idx], srcv)
                    np_vec = np_vec + plsc.all_reduce_population_count(m)
                nps = lax.reduce_max(np_vec, axes=(0,))
                do = nps >= EGB

                @pl.when(do)
                def _():
                    accum_batch(phase)
                    pack_batch(woff)
                    shift_residual()

                return (jnp.where(do, np_vec - EGB, np_vec),
                        jnp.where(do, woff + EGB, woff))

            return lax.fori_loop(0, ENSUB, sub, carry)

        load_chunk(0, 0)

        def pairf(t, carry):
            load_chunk(1, 2 * t + 1)
            wait_chunk(0)
            carry = scan_chunk(0, carry)

            @pl.when(2 * t + 2 < ENCHK)
            def _():
                load_chunk(0, 2 * t + 2)

            wait_chunk(1)
            return scan_chunk(1, carry)

        np_vec, woff = lax.fori_loop(0, ENCHK // 2, pairf, (zero_i, jnp.int32(0)))
        for w in range(EGB // 16):
            pos = i16 + w * 16
            keep = pos < np_vec
            cur_dl = pend_dl[pl.ds(w * 16, 16)]
            cur_sr = pend_src[pl.ds(w * 16, 16)]
            pend_dl[pl.ds(w * 16, 16)] = jnp.where(keep, cur_dl, NBA)
            pend_src[pl.ds(w * 16, 16)] = jnp.where(keep, cur_sr, (r * 311) % N)
        accum_batch(phase)
        pack_batch(woff)
        pltpu.make_async_copy(sbuf.at[pl.ds(0, EGB)],
                              elist_hbm.at[pl.ds(0, EGB)], swsem).wait()
        trash = jnp.full((16,), NBA * 16384 + (3 * 311) % N, jnp.int32)
        for k in range(EGB // 16):
            sbuf[pl.ds(k * 16, 16)] = trash
        pltpu.sync_copy(sbuf.at[pl.ds(0, EGB)],
                        elist_hbm.at[pl.ds(r * EE + pl.multiple_of(woff, EGB) + EGB, EGB)])
        return woff // EGB + 1

    def run_replay():
        init_phase(1)
        nb2 = (nb + 1) // 2

        def batchf(b, carry):
            pltpu.sync_copy(elist_hbm.at[pl.ds(r * EE + b * 2 * EGB, 2 * EGB)], sbuf)
            for k in range(2 * EGB // 16):
                v = sbuf[pl.ds(k * 16, 16)]
                pend_dl[pl.ds(k * 16, 16)] = jnp.right_shift(v, 14)
                pend_src[pl.ds(k * 16, 16)] = jnp.bitwise_and(v, 16383)
            accum_batch(1, ngroups=2 * EGB // 16)
            return carry

        lax.fori_loop(0, nb2, batchf, 0)

    def write_out(phase, nrows):
        row0 = r * NBA
        if phase == 0:
            pltpu.sync_copy(acc1.at[pl.ds(0, nrows)], ssum_hbm.at[pl.ds(row0, nrows)])
            pltpu.sync_copy(acc2.at[pl.ds(0, nrows)], ssq_hbm.at[pl.ds(row0, nrows)])
            pltpu.sync_copy(dega.at[pl.ds(0, nrows * 16)],
                            deg_hbm.at[pl.ds(row0 * 16, nrows * 16)])
        else:
            pltpu.sync_copy(acc1.at[pl.ds(0, nrows)], mx_hbm.at[pl.ds(row0, nrows)])
            pltpu.sync_copy(acc2.at[pl.ds(0, nrows)], mn_hbm.at[pl.ds(row0, nrows)])

    def finish(phase):
        @pl.when(r < 31)
        def _():
            write_out(phase, NBA)

        @pl.when(r == 31)
        def _():
            write_out(phase, N - 31 * NBA)

    nb = run_phase(0)
    finish(0)
    run_replay()
    finish(1)


def _stage_a(dst, src, x):
    f = pl.kernel(
        _stage_a_body,
        out_type=[jax.ShapeDtypeStruct((N, D), jnp.float32),
                  jax.ShapeDtypeStruct((N, D), jnp.float32),
                  jax.ShapeDtypeStruct((N, D), jnp.float32),
                  jax.ShapeDtypeStruct((N, D), jnp.float32),
                  jax.ShapeDtypeStruct((N * 16,), jnp.float32),
                  jax.ShapeDtypeStruct((NW * EE,), jnp.int32)],
        mesh=_sc_mesh(),
        scratch_types=[pltpu.VMEM((ECH,), jnp.int32), pltpu.VMEM((ECH,), jnp.int32),
                       pltpu.VMEM((ECH,), jnp.int32), pltpu.VMEM((ECH,), jnp.int32),
                       pltpu.VMEM((272,), jnp.int32), pltpu.VMEM((272,), jnp.int32),
                       pltpu.VMEM((2 * EGB, D), jnp.float32),
                       pltpu.VMEM((NBA + 1, D), jnp.float32),
                       pltpu.VMEM((NBA + 1, D), jnp.float32),
                       pltpu.VMEM(((NBA + 8) * 16,), jnp.float32),
                       pltpu.VMEM((2 * EGB,), jnp.int32),
                       pltpu.SemaphoreType.DMA, pltpu.SemaphoreType.DMA,
                       pltpu.SemaphoreType.DMA, pltpu.SemaphoreType.DMA],
        compiler_params=pltpu.CompilerParams(needs_layout_passes=False),
    )
    return f(dst, src, x)[:5]


def _stage_c_body(t_hbm, pi0_hbm, pi1_hbm, z_hbm, part_hbm,
                  idx0a, idx1a, ura, vra, zba,
                  idx0b, idx1b, urb, vrb, zbb,
                  pbuf, gsa, gsb, zsa, zsb):
    c = lax.axis_index("c")
    s = lax.axis_index("s")
    wid = s * 2 + c
    base = wid * PW
    zero = jnp.zeros((16,), jnp.float32)
    bufs = ((idx0a, idx1a, ura, vra, zba, gsa, zsa),
            (idx0b, idx1b, urb, vrb, zbb, gsb, zsb))

    def start(p, off):
        idx0, idx1, urows, vrows, _, gsem, _ = bufs[p]
        pltpu.sync_copy(pi0_hbm.at[pl.ds(off, GC)], idx0)
        pltpu.sync_copy(pi1_hbm.at[pl.ds(off, GC)], idx1)
        pltpu.async_copy(t_hbm.at[idx0], urows, gsem)
        pltpu.async_copy(t_hbm.at[idx1], vrows, gsem)

    def wait_g(p):
        idx0, idx1, urows, vrows, _, gsem, _ = bufs[p]
        pltpu.make_async_copy(t_hbm.at[idx0], urows, gsem).wait()
        pltpu.make_async_copy(t_hbm.at[idx1], vrows, gsem).wait()

    def wait_z(p):
        zbuf, zsem = bufs[p][4], bufs[p][6]
        pltpu.make_async_copy(zbuf, z_hbm.at[pl.ds(0, GC)], zsem).wait()

    def compute(p, off, carry):
        _, _, urows, vrows, zbuf, _, zsem = bufs[p]

        def row(i, rc):
            rs0, rs1, rq0, rq1 = rc
            a0 = urows[i, pl.ds(0, 16)] + vrows[i, pl.ds(32, 16)]
            a1 = urows[i, pl.ds(16, 16)] + vrows[i, pl.ds(48, 16)]
            zbuf[i, pl.ds(0, 16)] = a0
            zbuf[i, pl.ds(16, 16)] = a1
            return (rs0 + a0, rs1 + a1, rq0 + a0 * a0, rq1 + a1 * a1)

        carry = lax.fori_loop(0, GC, row, carry)
        pltpu.async_copy(zbuf, z_hbm.at[pl.ds(off, GC)], zsem)
        return carry

    TPAIR = NCH // 2
    start(0, base)

    def pair(t, carry):
        off0 = base + (2 * t) * GC
        start(1, off0 + GC)
        wait_g(0)

        @pl.when(t > 0)
        def _():
            wait_z(0)

        carry = compute(0, off0, carry)
        start(0, off0 + 2 * GC)
        wait_g(1)

        @pl.when(t > 0)
        def _():
            wait_z(1)

        return compute(1, off0 + GC, carry)

    carry = lax.fori_loop(0, TPAIR, pair, (zero, zero, zero, zero))
    wait_g(0)
    wait_z(0)
    s0, s1, q0, q1 = compute(0, base + (NCH - 1) * GC, carry)
    wait_z(0)
    wait_z(1)
    pbuf[pl.ds(0, 16)] = s0
    pbuf[pl.ds(16, 16)] = s1
    pbuf[pl.ds(32, 16)] = q0
    pbuf[pl.ds(48, 16)] = q1
    pltpu.sync_copy(pbuf, part_hbm.at[wid])


def _stage_c(t, pi0, pi1):
    dbl = lambda: [pltpu.VMEM((GC,), jnp.int32), pltpu.VMEM((GC,), jnp.int32),
                   pltpu.VMEM((GC, D), jnp.float32), pltpu.VMEM((GC, D), jnp.float32),
                   pltpu.VMEM((GC, HID), jnp.float32)]
    f = pl.kernel(
        _stage_c_body,
        out_type=[jax.ShapeDtypeStruct((P, HID), jnp.float32),
                  jax.ShapeDtypeStruct((NW, 2 * HID), jnp.float32)],
        mesh=_sc_mesh(),
        scratch_types=dbl() + dbl() + [pltpu.VMEM((2 * HID,), jnp.float32),
                                       pltpu.SemaphoreType.DMA,
                                       pltpu.SemaphoreType.DMA,
                                       pltpu.SemaphoreType.DMA,
                                       pltpu.SemaphoreType.DMA],
        compiler_params=pltpu.CompilerParams(needs_layout_passes=False),
    )
    return f(t, pi0, pi1)


def kernel(x, edge_index, pairwise_indices, W_post, b_post, W1, b1, gamma, beta, W2, b2):
    src = edge_index[0]
    dst = edge_index[1]
    ssum, ssq, mx, mn, deg_flat = _stage_a(dst, src, x)
    deg = deg_flat.reshape(N, 16)[:, 0:1]
    delta = _delta(deg)
    t = _stage_b(x, ssum, ssq, mx, mn, deg, delta, W_post, b_post, W1)
    z, moments = _stage_c(t, pairwise_indices[0], pairwise_indices[1])
    return _stage_d(z, moments, gamma, beta, W2, b2)
